# Initial kernel scaffold; baseline (speedup 1.0000x reference)
import jax
import jax.numpy as jnp
from jax.experimental import pallas as pl


def kernel(features, edge_index, Wi, bi, Wg, bg, gamma, beta, Wo, bo):
    raise NotImplementedError("write your pallas kernel here")


# speedup vs baseline: 6.5605x; 6.5605x over previous
import functools

import jax
import jax.numpy as jnp
from jax import lax
from jax.experimental import pallas as pl
from jax.experimental.pallas import tpu as pltpu
from jax.experimental.pallas import tpu_sc as plsc

_NC = 2
_NS = 16
_NW = _NC * _NS
_EB = 128
_GRP = 8 * _EB


def _sc_mesh():
    return plsc.VectorSubcoreMesh(core_axis_name="c", subcore_axis_name="s",
                                  num_cores=_NC, num_subcores=_NS)


def _sc_degrees(idx4, zeros_nd, ones_nd, n_pad):
    g = idx4.shape[1]
    iters = (g + _NS - 1) // _NS
    npt = n_pad // _NS

    @functools.partial(
        pl.kernel,
        out_type=jax.ShapeDtypeStruct((_NC, n_pad, 128), jnp.float32),
        mesh=_sc_mesh(),
        scratch_types=[
            pltpu.VMEM((8, _EB), jnp.int32),
            pltpu.VMEM((_EB, 128), jnp.float32),
            pltpu.VMEM_SHARED((n_pad, 128), jnp.float32),
        ],
    )
    def k(idx_hbm, zeros_hbm, ones_hbm, out_hbm, idx, ones_v, acc):
        cid = lax.axis_index("c")
        sid = lax.axis_index("s")
        off = pl.multiple_of(sid * npt, 8)
        pltpu.sync_copy(ones_hbm, ones_v)
        pltpu.sync_copy(zeros_hbm, acc.at[pl.ds(off, npt)])
        plsc.subcore_barrier()

        def body(i, carry):
            grp = sid + i * _NS

            @pl.when(grp < g)
            def _():
                pltpu.sync_copy(idx_hbm.at[cid, grp], idx)
                for j in range(8):
                    pltpu.sync_copy(ones_v, acc.at[idx.at[j]], add=True)

            return carry

        lax.fori_loop(0, iters, body, 0)
        plsc.subcore_barrier()
        sl = pl.ds(off, npt)
        pltpu.sync_copy(acc.at[sl], out_hbm.at[cid, sl])

    return k(idx4, zeros_nd, ones_nd)


def _sc_spmm(h, src3, dst3, zeros_nd):
    n_pad, d = h.shape
    g = src3.shape[0]
    iters = (g + _NW - 1) // _NW
    npt = n_pad // _NS

    @functools.partial(
        pl.kernel,
        out_type=jax.ShapeDtypeStruct((_NC, n_pad, d), jnp.float32),
        mesh=_sc_mesh(),
        scratch_types=[
            pltpu.VMEM((8, _EB), jnp.int32),
            pltpu.VMEM((8, _EB), jnp.int32),
            pltpu.VMEM((_EB, 128), jnp.float32),
            pltpu.VMEM_SHARED((n_pad, 128), jnp.float32),
            pltpu.SemaphoreType.DMA,
        ],
    )
    def k(h_hbm, src_hbm, dst_hbm, zeros_hbm, out_hbm,
          sidx, didx, rows, acc, sem):
        cid = lax.axis_index("c")
        sid = lax.axis_index("s")
        wid = sid * _NC + cid
        off = pl.multiple_of(sid * npt, 8)
        pltpu.sync_copy(zeros_hbm, acc.at[pl.ds(off, npt)])
        plsc.subcore_barrier()

        def body(i, carry):
            grp = wid + i * _NW

            @pl.when(grp < g)
            def _():
                pltpu.sync_copy(src_hbm.at[grp], sidx)
                pltpu.sync_copy(dst_hbm.at[grp], didx)
                for j in range(8):
                    pltpu.async_copy(h_hbm.at[sidx.at[j]], rows, sem).wait()
                    pltpu.sync_copy(rows, acc.at[didx.at[j]], add=True)

            return carry

        lax.fori_loop(0, iters, body, 0)
        plsc.subcore_barrier()
        sl = pl.ds(off, npt)
        pltpu.sync_copy(acc.at[sl], out_hbm.at[cid, sl])

    return k(h, src3, dst3, zeros_nd)


def _norms_from_deg(degp_ref, which):
    deg = degp_ref[which, :, 0]
    return jnp.where(deg > 0.0, lax.rsqrt(deg), 0.0)


def _gelu_exact(x):
    return x * 0.5 * (1.0 + lax.erf(x * 0.7071067811865476))


def _tc_inproj(features, wi, bi, degp, bn):
    n_pad, d = features.shape

    def body(f_ref, w_ref, b_ref, degp_ref, x_ref, h_ref):
        x = jnp.dot(f_ref[...], w_ref[...],
                    preferred_element_type=jnp.float32) + b_ref[...]
        norm_o = _norms_from_deg(degp_ref, 0)
        x_ref[...] = x
        h_ref[...] = x * norm_o[:, None]

    return pl.pallas_call(
        body,
        grid=(n_pad // bn,),
        in_specs=[
            pl.BlockSpec((bn, d), lambda i: (i, 0)),
            pl.BlockSpec((d, d), lambda i: (0, 0)),
            pl.BlockSpec((1, d), lambda i: (0, 0)),
            pl.BlockSpec((_NC, bn, 128), lambda i: (0, i, 0)),
        ],
        out_specs=[
            pl.BlockSpec((bn, d), lambda i: (i, 0)),
            pl.BlockSpec((bn, d), lambda i: (i, 0)),
        ],
        out_shape=[
            jax.ShapeDtypeStruct((n_pad, d), jnp.float32),
            jax.ShapeDtypeStruct((n_pad, d), jnp.float32),
        ],
    )(features, wi, bi, degp)


def _tc_layer(p, degp, xprev, w, b, g, bt, bn):
    _, n_pad, d = p.shape

    def body(p_ref, degp_ref, x_ref, w_ref, b_ref, g_ref, bt_ref,
             xo_ref, ho_ref):
        norm_i = _norms_from_deg(degp_ref, 1)
        agg = (p_ref[0] + p_ref[1]) * norm_i[:, None]
        t = jnp.dot(agg, w_ref[...],
                    preferred_element_type=jnp.float32) + b_ref[...]
        t = _gelu_exact(t)
        mu = jnp.mean(t, axis=-1, keepdims=True)
        var = jnp.mean((t - mu) ** 2, axis=-1, keepdims=True)
        t = (t - mu) * lax.rsqrt(var + 1e-5) * g_ref[...] + bt_ref[...]
        x = t + x_ref[...]
        xo_ref[...] = x
        norm_o = _norms_from_deg(degp_ref, 0)
        ho_ref[...] = x * norm_o[:, None]

    return pl.pallas_call(
        body,
        grid=(n_pad // bn,),
        in_specs=[
            pl.BlockSpec((_NC, bn, d), lambda i: (0, i, 0)),
            pl.BlockSpec((_NC, bn, 128), lambda i: (0, i, 0)),
            pl.BlockSpec((bn, d), lambda i: (i, 0)),
            pl.BlockSpec((d, d), lambda i: (0, 0)),
            pl.BlockSpec((1, d), lambda i: (0, 0)),
            pl.BlockSpec((1, d), lambda i: (0, 0)),
            pl.BlockSpec((1, d), lambda i: (0, 0)),
        ],
        out_specs=[
            pl.BlockSpec((bn, d), lambda i: (i, 0)),
            pl.BlockSpec((bn, d), lambda i: (i, 0)),
        ],
        out_shape=[
            jax.ShapeDtypeStruct((n_pad, d), jnp.float32),
            jax.ShapeDtypeStruct((n_pad, d), jnp.float32),
        ],
    )(p, degp, xprev, w, b, g, bt)


def _tc_layer_last(p, degp, xprev, w, b, g, bt, wo, bo, bn):
    _, n_pad, d = p.shape

    def body(p_ref, degp_ref, x_ref, w_ref, b_ref, g_ref, bt_ref,
             wo_ref, bo_ref, o_ref):
        norm_i = _norms_from_deg(degp_ref, 1)
        agg = (p_ref[0] + p_ref[1]) * norm_i[:, None]
        t = jnp.dot(agg, w_ref[...],
                    preferred_element_type=jnp.float32) + b_ref[...]
        t = _gelu_exact(t)
        mu = jnp.mean(t, axis=-1, keepdims=True)
        var = jnp.mean((t - mu) ** 2, axis=-1, keepdims=True)
        t = (t - mu) * lax.rsqrt(var + 1e-5) * g_ref[...] + bt_ref[...]
        x = t + x_ref[...]
        o_ref[...] = jnp.dot(x, wo_ref[...],
                             preferred_element_type=jnp.float32) + bo_ref[...]

    return pl.pallas_call(
        body,
        grid=(n_pad // bn,),
        in_specs=[
            pl.BlockSpec((_NC, bn, d), lambda i: (0, i, 0)),
            pl.BlockSpec((_NC, bn, 128), lambda i: (0, i, 0)),
            pl.BlockSpec((bn, d), lambda i: (i, 0)),
            pl.BlockSpec((d, d), lambda i: (0, 0)),
            pl.BlockSpec((1, d), lambda i: (0, 0)),
            pl.BlockSpec((1, d), lambda i: (0, 0)),
            pl.BlockSpec((1, d), lambda i: (0, 0)),
            pl.BlockSpec((d, d), lambda i: (0, 0)),
            pl.BlockSpec((1, d), lambda i: (0, 0)),
        ],
        out_specs=pl.BlockSpec((bn, d), lambda i: (i, 0)),
        out_shape=jax.ShapeDtypeStruct((n_pad, d), jnp.float32),
    )(p, degp, xprev, w, b, g, bt, wo, bo)


def kernel(features, edge_index, Wi, bi, Wg, bg, gamma, beta, Wo, bo):
    n, d = features.shape
    e = edge_index.shape[1]
    n_pad = ((n + 127) // 128) * 128
    e_pad = ((e + _GRP - 1) // _GRP) * _GRP
    g = e_pad // _GRP
    bn = 1264

    feats_p = jnp.pad(features, ((0, n_pad - n), (0, 0)))
    pad_idx = jnp.full((e_pad - e,), n, jnp.int32)
    src3 = jnp.concatenate([edge_index[0], pad_idx]).reshape(g, 8, _EB)
    dst3 = jnp.concatenate([edge_index[1], pad_idx]).reshape(g, 8, _EB)
    zeros_nd = jnp.zeros((n_pad // _NS, 128), jnp.float32)
    ones_nd = jnp.ones((_EB, 128), jnp.float32)
    bi2 = bi.reshape(1, d)
    bo2 = bo.reshape(1, d)

    idx4 = jnp.stack([src3, dst3])
    degp = _sc_degrees(idx4, zeros_nd, ones_nd, n_pad)
    x, h = _tc_inproj(feats_p, Wi, bi2, degp, bn)
    num_layers = Wg.shape[0]
    for l in range(num_layers):
        p = _sc_spmm(h, src3, dst3, zeros_nd)
        if l < num_layers - 1:
            x, h = _tc_layer(p, degp, x, Wg[l], bg[l].reshape(1, d),
                             gamma[l].reshape(1, d), beta[l].reshape(1, d), bn)
        else:
            out = _tc_layer_last(p, degp, x, Wg[l], bg[l].reshape(1, d),
                                 gamma[l].reshape(1, d), beta[l].reshape(1, d),
                                 Wo, bo2, bn)
    return out[:n]


# speedup vs baseline: 10.6752x; 1.6272x over previous
import functools

import jax
import jax.numpy as jnp
from jax import lax
from jax.experimental import pallas as pl
from jax.experimental.pallas import tpu as pltpu
from jax.experimental.pallas import tpu_sc as plsc

_NC = 2
_NS = 16
_NW = _NC * _NS
_EB = 128
_GRP = 8 * _EB


def _sc_mesh():
    return plsc.VectorSubcoreMesh(core_axis_name="c", subcore_axis_name="s",
                                  num_cores=_NC, num_subcores=_NS)


def _sc_degrees(idx4, zeros_hist, n_hist):
    g = idx4.shape[1]
    iters = (g + _NS - 1) // _NS
    npt = n_hist // _NS

    @functools.partial(
        pl.kernel,
        out_type=jax.ShapeDtypeStruct((_NC, 1, n_hist), jnp.float32),
        mesh=_sc_mesh(),
        compiler_params=pltpu.CompilerParams(needs_layout_passes=False),
        scratch_types=[
            pltpu.VMEM((8, _EB), jnp.int32),
            pltpu.VMEM((n_hist,), jnp.float32),
            pltpu.VMEM((_NS, 1, npt), jnp.float32),
            pltpu.VMEM_SHARED((_NS, 1, n_hist), jnp.float32),
        ],
    )
    def k(idx_hbm, zeros_hbm, out_hbm, idxg, hist, red, stage):
        cid = lax.axis_index("c")
        sid = lax.axis_index("s")
        pltpu.sync_copy(zeros_hbm, hist)
        ones = jnp.ones((16,), jnp.float32)

        def body(i, carry):
            grp = sid + i * _NS

            @pl.when(grp < g)
            def _():
                pltpu.sync_copy(idx_hbm.at[cid, grp], idxg)
                for j in range(8):
                    for v in range(8):
                        ii = idxg[j, pl.ds(v * 16, 16)]
                        plsc.addupdate_scatter(hist, [ii], ones)

            return carry

        lax.fori_loop(0, iters, body, 0)
        pltpu.sync_copy(hist, stage.at[sid, 0])
        plsc.subcore_barrier()
        off = pl.multiple_of(sid * npt, 128)
        pltpu.sync_copy(stage.at[:, :, pl.ds(off, npt)], red)
        for v in range(npt // 16):
            acc16 = red[0, 0, pl.ds(v * 16, 16)]
            for t in range(1, _NS):
                acc16 = acc16 + red[t, 0, pl.ds(v * 16, 16)]
            red[0, 0, pl.ds(v * 16, 16)] = acc16
        pltpu.sync_copy(red.at[0], out_hbm.at[cid, :, pl.ds(off, npt)])

    return k(idx4, zeros_hist)


def _sc_spmm(h, idx4, zeros_nd):
    n_pad, d = h.shape
    g = idx4.shape[1]
    iters = (g + _NW - 1) // _NW
    npt = n_pad // _NS

    @functools.partial(
        pl.kernel,
        out_type=jax.ShapeDtypeStruct((_NC, n_pad, d), jnp.float32),
        mesh=_sc_mesh(),
        scratch_types=[
            pltpu.VMEM((2, 2, 8, _EB), jnp.int32),
            pltpu.VMEM((_EB, 128), jnp.float32),
            pltpu.VMEM((_EB, 128), jnp.float32),
            pltpu.VMEM_SHARED((n_pad, 128), jnp.float32),
            pltpu.SemaphoreType.DMA,
            pltpu.SemaphoreType.DMA,
            pltpu.SemaphoreType.DMA,
            pltpu.SemaphoreType.DMA,
        ],
    )
    def k(h_hbm, idx_hbm, zeros_hbm, out_hbm,
          idxv, rows_a, rows_b, acc, sem_i, sem_g, ss0, ss1):
        cid = lax.axis_index("c")
        sid = lax.axis_index("s")
        wid = sid * _NC + cid
        off = pl.multiple_of(sid * npt, 8)
        pltpu.sync_copy(zeros_hbm, acc.at[pl.ds(off, npt)])

        @pl.when(wid < g)
        def _():
            pltpu.async_copy(idx_hbm.at[0, wid], idxv.at[0, 0], sem_i)
            pltpu.async_copy(idx_hbm.at[1, wid], idxv.at[0, 1], sem_i)

        plsc.subcore_barrier()
        bufs = (rows_a, rows_b)
        ssems = (ss0, ss1)

        def body(i, carry):
            slot = lax.rem(i, 2)
            grp = wid + i * _NW

            @pl.when(grp < g)
            def _():
                pltpu.make_async_copy(idx_hbm.at[0, grp],
                                      idxv.at[slot, 0], sem_i).wait()
                pltpu.make_async_copy(idx_hbm.at[1, grp],
                                      idxv.at[slot, 1], sem_i).wait()

                @pl.when(grp + _NW < g)
                def _():
                    pltpu.async_copy(idx_hbm.at[0, grp + _NW],
                                     idxv.at[1 - slot, 0], sem_i)
                    pltpu.async_copy(idx_hbm.at[1, grp + _NW],
                                     idxv.at[1 - slot, 1], sem_i)

                def q_gathers(j, buf):
                    return [pltpu.async_copy(
                        h_hbm.at[idxv.at[slot, 0, j, pl.ds(q * 32, 32)]],
                        buf.at[pl.ds(q * 32, 32)], sem_g) for q in range(4)]

                pend_g = q_gathers(0, bufs[0])
                prev_s = None
                for j in range(8):
                    for dsc in pend_g:
                        dsc.wait()
                    if prev_s is not None:
                        prev_s.wait()
                    if j < 7:
                        nxt_g = q_gathers(j + 1, bufs[(j + 1) % 2])
                    prev_s = pltpu.async_copy(bufs[j % 2],
                                              acc.at[idxv.at[slot, 1, j]],
                                              ssems[j % 2], add=True)
                    if j < 7:
                        pend_g = nxt_g
                prev_s.wait()

            return carry

        lax.fori_loop(0, iters, body, 0)
        plsc.subcore_barrier()
        sl = pl.ds(off, npt)
        pltpu.sync_copy(acc.at[sl], out_hbm.at[cid, sl])

    return k(h, idx4, zeros_nd)


def _norms_from_deg(degp_ref, which):
    deg = degp_ref[which, 0, :]
    return jnp.where(deg > 0.0, lax.rsqrt(deg), 0.0)


def _gelu_exact(x):
    return x * 0.5 * (1.0 + lax.erf(x * 0.7071067811865476))


def _tc_inproj(features, wi, bi, degp, bn):
    n_pad, d = features.shape

    def body(f_ref, w_ref, b_ref, degp_ref, x_ref, h_ref):
        x = jnp.dot(f_ref[...], w_ref[...],
                    preferred_element_type=jnp.float32) + b_ref[...]
        norm_o = _norms_from_deg(degp_ref, 0)
        x_ref[...] = x
        h_ref[...] = x * norm_o[:, None]

    return pl.pallas_call(
        body,
        grid=(n_pad // bn,),
        in_specs=[
            pl.BlockSpec((bn, d), lambda i: (i, 0)),
            pl.BlockSpec((d, d), lambda i: (0, 0)),
            pl.BlockSpec((1, d), lambda i: (0, 0)),
            pl.BlockSpec((_NC, 1, bn), lambda i: (0, 0, i)),
        ],
        out_specs=[
            pl.BlockSpec((bn, d), lambda i: (i, 0)),
            pl.BlockSpec((bn, d), lambda i: (i, 0)),
        ],
        out_shape=[
            jax.ShapeDtypeStruct((n_pad, d), jnp.float32),
            jax.ShapeDtypeStruct((n_pad, d), jnp.float32),
        ],
    )(features, wi, bi, degp)


def _tc_layer(p, degp, xprev, w, b, g, bt, bn):
    _, n_pad, d = p.shape

    def body(p_ref, degp_ref, x_ref, w_ref, b_ref, g_ref, bt_ref,
             xo_ref, ho_ref):
        norm_i = _norms_from_deg(degp_ref, 1)
        agg = (p_ref[0] + p_ref[1]) * norm_i[:, None]
        t = jnp.dot(agg, w_ref[...],
                    preferred_element_type=jnp.float32) + b_ref[...]
        t = _gelu_exact(t)
        mu = jnp.mean(t, axis=-1, keepdims=True)
        var = jnp.mean((t - mu) ** 2, axis=-1, keepdims=True)
        t = (t - mu) * lax.rsqrt(var + 1e-5) * g_ref[...] + bt_ref[...]
        x = t + x_ref[...]
        xo_ref[...] = x
        norm_o = _norms_from_deg(degp_ref, 0)
        ho_ref[...] = x * norm_o[:, None]

    return pl.pallas_call(
        body,
        grid=(n_pad // bn,),
        in_specs=[
            pl.BlockSpec((_NC, bn, d), lambda i: (0, i, 0)),
            pl.BlockSpec((_NC, 1, bn), lambda i: (0, 0, i)),
            pl.BlockSpec((bn, d), lambda i: (i, 0)),
            pl.BlockSpec((d, d), lambda i: (0, 0)),
            pl.BlockSpec((1, d), lambda i: (0, 0)),
            pl.BlockSpec((1, d), lambda i: (0, 0)),
            pl.BlockSpec((1, d), lambda i: (0, 0)),
        ],
        out_specs=[
            pl.BlockSpec((bn, d), lambda i: (i, 0)),
            pl.BlockSpec((bn, d), lambda i: (i, 0)),
        ],
        out_shape=[
            jax.ShapeDtypeStruct((n_pad, d), jnp.float32),
            jax.ShapeDtypeStruct((n_pad, d), jnp.float32),
        ],
    )(p, degp, xprev, w, b, g, bt)


def _tc_layer_last(p, degp, xprev, w, b, g, bt, wo, bo, bn):
    _, n_pad, d = p.shape

    def body(p_ref, degp_ref, x_ref, w_ref, b_ref, g_ref, bt_ref,
             wo_ref, bo_ref, o_ref):
        norm_i = _norms_from_deg(degp_ref, 1)
        agg = (p_ref[0] + p_ref[1]) * norm_i[:, None]
        t = jnp.dot(agg, w_ref[...],
                    preferred_element_type=jnp.float32) + b_ref[...]
        t = _gelu_exact(t)
        mu = jnp.mean(t, axis=-1, keepdims=True)
        var = jnp.mean((t - mu) ** 2, axis=-1, keepdims=True)
        t = (t - mu) * lax.rsqrt(var + 1e-5) * g_ref[...] + bt_ref[...]
        x = t + x_ref[...]
        o_ref[...] = jnp.dot(x, wo_ref[...],
                             preferred_element_type=jnp.float32) + bo_ref[...]

    return pl.pallas_call(
        body,
        grid=(n_pad // bn,),
        in_specs=[
            pl.BlockSpec((_NC, bn, d), lambda i: (0, i, 0)),
            pl.BlockSpec((_NC, 1, bn), lambda i: (0, 0, i)),
            pl.BlockSpec((bn, d), lambda i: (i, 0)),
            pl.BlockSpec((d, d), lambda i: (0, 0)),
            pl.BlockSpec((1, d), lambda i: (0, 0)),
            pl.BlockSpec((1, d), lambda i: (0, 0)),
            pl.BlockSpec((1, d), lambda i: (0, 0)),
            pl.BlockSpec((d, d), lambda i: (0, 0)),
            pl.BlockSpec((1, d), lambda i: (0, 0)),
        ],
        out_specs=pl.BlockSpec((bn, d), lambda i: (i, 0)),
        out_shape=jax.ShapeDtypeStruct((n_pad, d), jnp.float32),
    )(p, degp, xprev, w, b, g, bt, wo, bo)


def kernel(features, edge_index, Wi, bi, Wg, bg, gamma, beta, Wo, bo):
    n, d = features.shape
    e = edge_index.shape[1]
    n_pad = ((n + 2047) // 2048) * 2048
    e_pad = ((e + _GRP - 1) // _GRP) * _GRP
    g = e_pad // _GRP
    bn = n_pad // 8

    feats_p = jnp.pad(features, ((0, n_pad - n), (0, 0)))
    pad_idx = n + jnp.arange(e_pad - e, dtype=jnp.int32) % (n_pad - n)
    src3 = jnp.concatenate([edge_index[0], pad_idx]).reshape(g, 8, _EB)
    dst3 = jnp.concatenate([edge_index[1], pad_idx]).reshape(g, 8, _EB)
    zeros_nd = jnp.zeros((n_pad // _NS, 128), jnp.float32)
    bi2 = bi.reshape(1, d)
    bo2 = bo.reshape(1, d)

    zeros_hist = jnp.zeros((n_pad,), jnp.float32)
    idx4 = jnp.stack([src3, dst3])
    degp = _sc_degrees(idx4, zeros_hist, n_pad)
    x, h = _tc_inproj(feats_p, Wi, bi2, degp, bn)
    num_layers = Wg.shape[0]
    for l in range(num_layers):
        p = _sc_spmm(h, idx4, zeros_nd)
        if l < num_layers - 1:
            x, h = _tc_layer(p, degp, x, Wg[l], bg[l].reshape(1, d),
                             gamma[l].reshape(1, d), beta[l].reshape(1, d), bn)
        else:
            out = _tc_layer_last(p, degp, x, Wg[l], bg[l].reshape(1, d),
                                 gamma[l].reshape(1, d), beta[l].reshape(1, d),
                                 Wo, bo2, bn)
    return out[:n]
